# Initial kernel scaffold; baseline (speedup 1.0000x reference)
import jax
import jax.numpy as jnp
from jax.experimental import pallas as pl


def kernel(input_ids, attention_mask, emb, W, b):
    raise NotImplementedError("write your pallas kernel here")


# speedup vs baseline: 17.2204x; 17.2204x over previous
import functools

import jax
import jax.numpy as jnp
from jax.experimental import pallas as pl
from jax.experimental.pallas import tpu as pltpu

_RB = 64


def _body(l, v, ids_ref, emb_ref, W_ref, b_ref, embeds_ref, pooled_ref):
    ids = ids_ref[0]
    tb = ids.shape[1]
    rb = tb // l
    iota_v = jax.lax.broadcasted_iota(jnp.int32, (v, tb), 0)
    onehot_t = (iota_v == ids).astype(jnp.float32)
    emb = emb_ref[...]
    embeds = jax.lax.dot_general(
        onehot_t, emb, (((0,), (0,)), ((), ())),
        preferred_element_type=jnp.float32)
    embeds_ref[...] = embeds
    t_iota = jax.lax.broadcasted_iota(jnp.int32, (tb, rb), 0)
    r_iota = jax.lax.broadcasted_iota(jnp.int32, (tb, rb), 1)
    seg = (t_iota // l == r_iota).astype(jnp.float32)
    counts_t = jax.lax.dot_general(
        onehot_t, seg, (((1,), (0,)), ((), ())),
        preferred_element_type=jnp.float32)
    pooled = jax.lax.dot_general(
        counts_t, emb, (((0,), (0,)), ((), ())),
        preferred_element_type=jnp.float32) * (1.0 / l)
    pooled = jax.lax.dot_general(
        pooled, W_ref[...], (((1,), (1,)), ((), ())),
        preferred_element_type=jnp.float32)
    pooled_ref[...] = pooled + b_ref[...]


@jax.jit
def kernel(input_ids, attention_mask, emb, W, b):
    del attention_mask
    bsz, l = input_ids.shape
    v, h = emb.shape
    nb = bsz // _RB
    tb = _RB * l
    ids3 = input_ids.astype(jnp.int32).reshape(nb, 1, tb)
    embeds2, pooled = pl.pallas_call(
        functools.partial(_body, l, v),
        grid=(nb,),
        in_specs=[
            pl.BlockSpec((1, 1, tb), lambda i: (i, 0, 0)),
            pl.BlockSpec((v, h), lambda i: (0, 0)),
            pl.BlockSpec((h, h), lambda i: (0, 0)),
            pl.BlockSpec((1, h), lambda i: (0, 0)),
        ],
        out_specs=[
            pl.BlockSpec((tb, h), lambda i: (i, 0)),
            pl.BlockSpec((_RB, h), lambda i: (i, 0)),
        ],
        out_shape=[
            jax.ShapeDtypeStruct((bsz * l, h), jnp.float32),
            jax.ShapeDtypeStruct((bsz, h), jnp.float32),
        ],
    )(ids3, emb, W, b.reshape(1, h))
    return (pooled, embeds2.reshape(bsz, l, h))
